# Initial kernel scaffold; baseline (speedup 1.0000x reference)
import jax
import jax.numpy as jnp
from jax.experimental import pallas as pl


def kernel(x, edge_index, W1, b1, W2, b2, Wg, bg, Wq, bq):
    raise NotImplementedError("write your pallas kernel here")


# speedup vs baseline: 14.3992x; 14.3992x over previous
import functools

import jax
import jax.numpy as jnp
from jax import lax
from jax.experimental import pallas as pl
from jax.experimental.pallas import tpu as pltpu
from jax.experimental.pallas import tpu_sc as plsc

N = 10000
NPAD = 10240
D = 128
NACT = 10000
NC = 2
NS = 16
NW = NC * NS
K = 128
RPT = NPAD // NS
BM = 512

_MESH = plsc.VectorSubcoreMesh(core_axis_name="c", subcore_axis_name="s")


def _deg_call(dst, zeros128, nwin, ew):
    @functools.partial(
        pl.kernel,
        out_type=jax.ShapeDtypeStruct((NC, NPAD, D), jnp.float32),
        mesh=_MESH,
        scratch_types=[
            pltpu.VMEM_SHARED((NPAD, D), jnp.float32),
            pltpu.VMEM((K,), jnp.int32),
            pltpu.VMEM((K, D), jnp.float32),
        ],
    )
    def deg_kernel(dst_hbm, z_hbm, out_hbm, acc_sh, idx_v, ones_v):
        c = lax.axis_index("c")
        s = lax.axis_index("s")
        wid = c * NS + s

        @pl.loop(0, K)
        def _(i):
            for j in range(0, D, 16):
                ones_v[i, pl.ds(j, 16)] = jnp.ones((16,), jnp.float32)
        pltpu.sync_copy(z_hbm.at[pl.ds(s * RPT, RPT)],
                        acc_sh.at[pl.ds(s * RPT, RPT)])
        plsc.subcore_barrier()
        base = pl.multiple_of(wid * ew, 8)

        @pl.loop(0, nwin)
        def _(w):
            off = pl.multiple_of(base + w * K, 8)
            pltpu.sync_copy(dst_hbm.at[pl.ds(off, K)], idx_v)
            pltpu.sync_copy(ones_v, acc_sh.at[idx_v], add=True)

        plsc.subcore_barrier()
        pltpu.sync_copy(acc_sh.at[pl.ds(s * RPT, RPT)],
                        out_hbm.at[c, pl.ds(s * RPT, RPT)])

    return deg_kernel(dst, zeros128)


def _agg_call(src, dst, ps, zeros128, nwin, ew):
    @functools.partial(
        pl.kernel,
        out_type=jax.ShapeDtypeStruct((NC, NPAD, D), jnp.float32),
        mesh=_MESH,
        scratch_types=[
            pltpu.VMEM_SHARED((NPAD, D), jnp.float32),
            pltpu.VMEM((K,), jnp.int32),
            pltpu.VMEM((K,), jnp.int32),
            pltpu.VMEM((K, D), jnp.float32),
            pltpu.SemaphoreType.DMA,
        ],
    )
    def agg_kernel(src_hbm, dst_hbm, ps_hbm, z_hbm, out_hbm,
                   acc_sh, idxs_v, idxd_v, rows_v, sem):
        c = lax.axis_index("c")
        s = lax.axis_index("s")
        wid = c * NS + s
        pltpu.sync_copy(z_hbm.at[pl.ds(s * RPT, RPT)],
                        acc_sh.at[pl.ds(s * RPT, RPT)])
        plsc.subcore_barrier()
        base = pl.multiple_of(wid * ew, 8)

        @pl.loop(0, nwin)
        def _(w):
            off = pl.multiple_of(base + w * K, 8)
            pltpu.sync_copy(src_hbm.at[pl.ds(off, K)], idxs_v)
            pltpu.sync_copy(dst_hbm.at[pl.ds(off, K)], idxd_v)
            pltpu.async_copy(ps_hbm.at[idxs_v], rows_v, sem).wait()
            pltpu.sync_copy(rows_v, acc_sh.at[idxd_v], add=True)

        plsc.subcore_barrier()
        pltpu.sync_copy(acc_sh.at[pl.ds(s * RPT, RPT)],
                        out_hbm.at[c, pl.ds(s * RPT, RPT)])

    return agg_kernel(src, dst, ps, zeros128)


def _mm_body(x_ref, w_ref, o_ref):
    o_ref[...] = lax.dot_general(
        x_ref[...], w_ref[...], (((1,), (1,)), ((), ())),
        preferred_element_type=jnp.float32)


def _mm_call(xp, w):
    return pl.pallas_call(
        _mm_body,
        grid=(NPAD // BM,),
        in_specs=[
            pl.BlockSpec((BM, D), lambda i: (i, 0)),
            pl.BlockSpec((D, D), lambda i: (0, 0)),
        ],
        out_specs=pl.BlockSpec((BM, D), lambda i: (i, 0)),
        out_shape=jax.ShapeDtypeStruct((NPAD, D), jnp.float32),
    )(xp, w)


def _scale_body(d0_ref, d1_ref, p_ref, dinv_ref, ps_ref):
    deg = d0_ref[...] + d1_ref[...] + 1.0
    dinv = lax.rsqrt(deg)
    dinv_ref[...] = dinv
    ps_ref[...] = p_ref[...] * dinv


def _scale_call(d0, d1, p1):
    return pl.pallas_call(
        _scale_body,
        grid=(NPAD // BM,),
        in_specs=[
            pl.BlockSpec((BM, 1), lambda i: (i, 0)),
            pl.BlockSpec((BM, 1), lambda i: (i, 0)),
            pl.BlockSpec((BM, D), lambda i: (i, 0)),
        ],
        out_specs=[
            pl.BlockSpec((BM, 1), lambda i: (i, 0)),
            pl.BlockSpec((BM, D), lambda i: (i, 0)),
        ],
        out_shape=[
            jax.ShapeDtypeStruct((NPAD, 1), jnp.float32),
            jax.ShapeDtypeStruct((NPAD, D), jnp.float32),
        ],
    )(d0, d1, p1)


def _layer2_body(a0_ref, a1_ref, ps1_ref, dinv_ref, b1_ref, w2_ref, ps2_ref):
    h = dinv_ref[...] * (a0_ref[0] + a1_ref[0] + ps1_ref[...]) + b1_ref[...]
    h = jnp.maximum(h, 0.0)
    p2 = lax.dot_general(h, w2_ref[...], (((1,), (1,)), ((), ())),
                         preferred_element_type=jnp.float32)
    ps2_ref[...] = dinv_ref[...] * p2


def _layer2_call(agg1, ps1, dinv, b1r, w2):
    return pl.pallas_call(
        _layer2_body,
        grid=(NPAD // BM,),
        in_specs=[
            pl.BlockSpec((1, BM, D), lambda i: (0, i, 0)),
            pl.BlockSpec((1, BM, D), lambda i: (1, i, 0)),
            pl.BlockSpec((BM, D), lambda i: (i, 0)),
            pl.BlockSpec((BM, 1), lambda i: (i, 0)),
            pl.BlockSpec((1, D), lambda i: (0, 0)),
            pl.BlockSpec((D, D), lambda i: (0, 0)),
        ],
        out_specs=pl.BlockSpec((BM, D), lambda i: (i, 0)),
        out_shape=jax.ShapeDtypeStruct((NPAD, D), jnp.float32),
    )(agg1, agg1, ps1, dinv, b1r, w2)


def _reduce_body(a0_ref, a1_ref, ps2_ref, dinv_ref, b2_ref, out_ref):
    i = pl.program_id(0)
    h = dinv_ref[...] * (a0_ref[0] + a1_ref[0] + ps2_ref[...]) + b2_ref[...]
    h = jnp.maximum(h, 0.0)
    rows = lax.broadcasted_iota(jnp.int32, (BM, 1), 0) + i * BM
    h = jnp.where(rows < N, h, 0.0)

    @pl.when(i == 0)
    def _():
        out_ref[...] = jnp.zeros_like(out_ref)

    out_ref[...] += jnp.sum(h, axis=0, keepdims=True)


def _reduce_call(agg2, ps2, dinv, b2r):
    return pl.pallas_call(
        _reduce_body,
        grid=(NPAD // BM,),
        in_specs=[
            pl.BlockSpec((1, BM, D), lambda i: (0, i, 0)),
            pl.BlockSpec((1, BM, D), lambda i: (1, i, 0)),
            pl.BlockSpec((BM, D), lambda i: (i, 0)),
            pl.BlockSpec((BM, 1), lambda i: (i, 0)),
            pl.BlockSpec((1, D), lambda i: (0, 0)),
        ],
        out_specs=pl.BlockSpec((1, D), lambda i: (0, 0)),
        out_shape=jax.ShapeDtypeStruct((1, D), jnp.float32),
    )(agg2, agg2, ps2, dinv, b2r)


def _head_body(g_ref, wg_ref, bg_ref, wq_ref, bq_ref, q_ref):
    g = g_ref[...] * (1.0 / N)
    gg = lax.dot_general(g, wg_ref[...], (((1,), (1,)), ((), ())),
                         preferred_element_type=jnp.float32) + bg_ref[...]
    gg = jnp.maximum(gg, 0.0)
    q_ref[...] = lax.dot_general(gg, wq_ref[...], (((1,), (1,)), ((), ())),
                                 preferred_element_type=jnp.float32) + bq_ref[...]


def _head_call(gsum, wg, bgr, wq, bqr):
    return pl.pallas_call(
        _head_body,
        out_shape=jax.ShapeDtypeStruct((1, NACT), jnp.float32),
    )(gsum, wg, bgr, wq, bqr)


def kernel(x, edge_index, W1, b1, W2, b2, Wg, bg, Wq, bq):
    ei = edge_index.astype(jnp.int32)
    src, dst = ei[0], ei[1]
    e = src.shape[0]
    epad = -(-e // (NW * K)) * (NW * K)
    ew = epad // NW
    nwin = ew // K
    npad_e = epad - e
    if npad_e:
        pidx = jnp.arange(npad_e, dtype=jnp.int32)
        src = jnp.concatenate([src, (pidx * 97) % N])
        dst = jnp.concatenate([dst, N + pidx % (NPAD - N)])

    xp = jnp.pad(x, ((0, NPAD - N), (0, 0)))
    zeros128 = jnp.zeros((NPAD, D), jnp.float32)

    degparts = _deg_call(dst, zeros128, nwin, ew)
    p1 = _mm_call(xp, W1)
    d0 = degparts[0, :, 0:1]
    d1 = degparts[1, :, 0:1]
    dinv, ps1 = _scale_call(d0, d1, p1)
    agg1 = _agg_call(src, dst, ps1, zeros128, nwin, ew)
    ps2 = _layer2_call(agg1, ps1, dinv, b1.reshape(1, D), W2)
    agg2 = _agg_call(src, dst, ps2, zeros128, nwin, ew)
    gsum = _reduce_call(agg2, ps2, dinv, b2.reshape(1, D))
    q = _head_call(gsum, Wg, bg.reshape(1, D), Wq, bq.reshape(1, NACT))
    return q[None]


# speedup vs baseline: 28.9105x; 2.0078x over previous
import dataclasses
import functools

import jax
import jax.numpy as jnp
from jax import lax
from jax.experimental import pallas as pl
from jax.experimental.pallas import tpu as pltpu
from jax.experimental.pallas import tpu_sc as plsc

N = 10000
NPAD = 10240
D = 128
NACT = 10000
NC = 2
NS = 16
NW = NC * NS
K = 128
NBUF = 2
CW = 16
RPT = NPAD // NS
BM = 512

_MESH = plsc.VectorSubcoreMesh(core_axis_name="c", subcore_axis_name="s")

_CP = pltpu.CompilerParams()
if "needs_layout_passes" in pltpu.CompilerParams.__dataclass_fields__:
    _CP = dataclasses.replace(_CP, needs_layout_passes=False)


def _deg_call(dst2, nwpw):
    @functools.partial(
        pl.kernel,
        out_type=jax.ShapeDtypeStruct((NC, NPAD), jnp.float32),
        mesh=_MESH,
        scratch_types=[
            pltpu.VMEM_SHARED((NS, NPAD), jnp.float32),
            pltpu.VMEM((NPAD,), jnp.float32),
            pltpu.VMEM((nwpw, K), jnp.int32),
            pltpu.VMEM((NS, RPT), jnp.float32),
            pltpu.VMEM((RPT,), jnp.float32),
        ],
        compiler_params=_CP,
    )
    def deg_kernel(dst_hbm, out_hbm, acc_sh, degl, idxd_v, stage_v, res_v):
        c = lax.axis_index("c")
        s = lax.axis_index("s")
        wid = c * NS + s

        @pl.loop(0, NPAD // 16)
        def _(i):
            degl[pl.ds(i * 16, 16)] = jnp.zeros((16,), jnp.float32)

        wbase = pl.multiple_of(wid * nwpw, 8)
        pltpu.sync_copy(dst_hbm.at[pl.ds(wbase, nwpw)], idxd_v)
        ones16 = jnp.ones((16,), jnp.float32)

        @pl.loop(0, nwpw)
        def _(w):
            for j in range(0, K, 16):
                idx16 = idxd_v[w, pl.ds(j, 16)]
                plsc.addupdate_scatter(degl, [idx16], ones16)

        pltpu.sync_copy(degl, acc_sh.at[s])
        plsc.subcore_barrier()
        pltpu.sync_copy(acc_sh.at[:, pl.ds(s * RPT, RPT)], stage_v)

        @pl.loop(0, RPT // 16)
        def _(i):
            acc = jnp.zeros((16,), jnp.float32)
            for t in range(NS):
                acc = acc + stage_v[t, pl.ds(i * 16, 16)]
            res_v[pl.ds(i * 16, 16)] = acc

        pltpu.sync_copy(res_v, out_hbm.at[c, pl.ds(s * RPT, RPT)])

    return deg_kernel(dst2)


def _agg_call(src2, dst2, ps, nwpw):
    @functools.partial(
        pl.kernel,
        out_type=jax.ShapeDtypeStruct((NC, NPAD, D), jnp.float32),
        mesh=_MESH,
        scratch_types=[
            pltpu.VMEM_SHARED((NPAD, D), jnp.float32),
            pltpu.VMEM((CW, K), jnp.int32),
            pltpu.VMEM((CW, K), jnp.int32),
            pltpu.VMEM((NBUF, K, D), jnp.float32),
            pltpu.SemaphoreType.DMA,
            pltpu.SemaphoreType.DMA,
        ],
    )
    def agg_kernel(src_hbm, dst_hbm, ps_hbm, out_hbm,
                   acc_sh, idxs_v, idxd_v, bufs, s0, s1):
        sems = [s0, s1]
        c = lax.axis_index("c")
        s = lax.axis_index("s")
        wid = c * NS + s
        wbase = pl.multiple_of(wid * nwpw, 8)

        @pl.loop(0, K)
        def _(i):
            for j in range(0, D, 16):
                bufs[0, i, pl.ds(j, 16)] = jnp.zeros((16,), jnp.float32)

        for r in range(RPT // K):
            pltpu.sync_copy(bufs.at[0], acc_sh.at[pl.ds(s * RPT + r * K, K)])
        plsc.subcore_barrier()

        @pl.loop(0, nwpw // CW)
        def _(kc):
            cbase = pl.multiple_of(wbase + kc * CW, 8)
            pltpu.sync_copy(src_hbm.at[pl.ds(cbase, CW)], idxs_v)
            pltpu.sync_copy(dst_hbm.at[pl.ds(cbase, CW)], idxd_v)
            for b in range(NBUF):
                pltpu.async_copy(ps_hbm.at[idxs_v.at[b]], bufs.at[b], sems[b])

            @pl.loop(0, CW // NBUF)
            def _(k):
                for b in range(NBUF):
                    w = k * NBUF + b
                    pltpu.make_async_copy(
                        ps_hbm.at[pl.ds(0, K)], bufs.at[b], sems[b]).wait()
                    pltpu.sync_copy(bufs.at[b], acc_sh.at[idxd_v.at[w]],
                                    add=True)
                    wn = w + NBUF

                    @pl.when(wn < CW)
                    def _():
                        pltpu.async_copy(ps_hbm.at[idxs_v.at[wn]], bufs.at[b],
                                         sems[b])

        plsc.subcore_barrier()
        pltpu.sync_copy(acc_sh.at[pl.ds(s * RPT, RPT)],
                        out_hbm.at[c, pl.ds(s * RPT, RPT)])

    return agg_kernel(src2, dst2, ps)


def _mm_body(x_ref, w_ref, o_ref):
    o_ref[...] = lax.dot_general(
        x_ref[...], w_ref[...], (((1,), (1,)), ((), ())),
        preferred_element_type=jnp.float32)


def _mm_call(xp, w):
    return pl.pallas_call(
        _mm_body,
        grid=(NPAD // BM,),
        in_specs=[
            pl.BlockSpec((BM, D), lambda i: (i, 0)),
            pl.BlockSpec((D, D), lambda i: (0, 0)),
        ],
        out_specs=pl.BlockSpec((BM, D), lambda i: (i, 0)),
        out_shape=jax.ShapeDtypeStruct((NPAD, D), jnp.float32),
    )(xp, w)


def _scale_body(d0_ref, d1_ref, p_ref, dinv_ref, ps_ref):
    deg = d0_ref[...] + d1_ref[...] + 1.0
    dinv = lax.rsqrt(deg)
    dinv_ref[...] = dinv
    ps_ref[...] = p_ref[...] * dinv


def _scale_call(d0, d1, p1):
    return pl.pallas_call(
        _scale_body,
        grid=(NPAD // BM,),
        in_specs=[
            pl.BlockSpec((BM, 1), lambda i: (i, 0)),
            pl.BlockSpec((BM, 1), lambda i: (i, 0)),
            pl.BlockSpec((BM, D), lambda i: (i, 0)),
        ],
        out_specs=[
            pl.BlockSpec((BM, 1), lambda i: (i, 0)),
            pl.BlockSpec((BM, D), lambda i: (i, 0)),
        ],
        out_shape=[
            jax.ShapeDtypeStruct((NPAD, 1), jnp.float32),
            jax.ShapeDtypeStruct((NPAD, D), jnp.float32),
        ],
    )(d0, d1, p1)


def _layer2_body(a0_ref, a1_ref, ps1_ref, dinv_ref, b1_ref, w2_ref, ps2_ref):
    h = dinv_ref[...] * (a0_ref[0] + a1_ref[0] + ps1_ref[...]) + b1_ref[...]
    h = jnp.maximum(h, 0.0)
    p2 = lax.dot_general(h, w2_ref[...], (((1,), (1,)), ((), ())),
                         preferred_element_type=jnp.float32)
    ps2_ref[...] = dinv_ref[...] * p2


def _layer2_call(agg1, ps1, dinv, b1r, w2):
    return pl.pallas_call(
        _layer2_body,
        grid=(NPAD // BM,),
        in_specs=[
            pl.BlockSpec((1, BM, D), lambda i: (0, i, 0)),
            pl.BlockSpec((1, BM, D), lambda i: (1, i, 0)),
            pl.BlockSpec((BM, D), lambda i: (i, 0)),
            pl.BlockSpec((BM, 1), lambda i: (i, 0)),
            pl.BlockSpec((1, D), lambda i: (0, 0)),
            pl.BlockSpec((D, D), lambda i: (0, 0)),
        ],
        out_specs=pl.BlockSpec((BM, D), lambda i: (i, 0)),
        out_shape=jax.ShapeDtypeStruct((NPAD, D), jnp.float32),
    )(agg1, agg1, ps1, dinv, b1r, w2)


def _reduce_body(a0_ref, a1_ref, ps2_ref, dinv_ref, b2_ref, out_ref):
    i = pl.program_id(0)
    h = dinv_ref[...] * (a0_ref[0] + a1_ref[0] + ps2_ref[...]) + b2_ref[...]
    h = jnp.maximum(h, 0.0)
    rows = lax.broadcasted_iota(jnp.int32, (BM, 1), 0) + i * BM
    h = jnp.where(rows < N, h, 0.0)

    @pl.when(i == 0)
    def _():
        out_ref[...] = jnp.zeros_like(out_ref)

    out_ref[...] += jnp.sum(h, axis=0, keepdims=True)


def _reduce_call(agg2, ps2, dinv, b2r):
    return pl.pallas_call(
        _reduce_body,
        grid=(NPAD // BM,),
        in_specs=[
            pl.BlockSpec((1, BM, D), lambda i: (0, i, 0)),
            pl.BlockSpec((1, BM, D), lambda i: (1, i, 0)),
            pl.BlockSpec((BM, D), lambda i: (i, 0)),
            pl.BlockSpec((BM, 1), lambda i: (i, 0)),
            pl.BlockSpec((1, D), lambda i: (0, 0)),
        ],
        out_specs=pl.BlockSpec((1, D), lambda i: (0, 0)),
        out_shape=jax.ShapeDtypeStruct((1, D), jnp.float32),
    )(agg2, agg2, ps2, dinv, b2r)


def _head_body(g_ref, wg_ref, bg_ref, wq_ref, bq_ref, q_ref):
    g = g_ref[...] * (1.0 / N)
    gg = lax.dot_general(g, wg_ref[...], (((1,), (1,)), ((), ())),
                         preferred_element_type=jnp.float32) + bg_ref[...]
    gg = jnp.maximum(gg, 0.0)
    q_ref[...] = lax.dot_general(gg, wq_ref[...], (((1,), (1,)), ((), ())),
                                 preferred_element_type=jnp.float32) + bq_ref[...]


def _head_call(gsum, wg, bgr, wq, bqr):
    return pl.pallas_call(
        _head_body,
        out_shape=jax.ShapeDtypeStruct((1, NACT), jnp.float32),
    )(gsum, wg, bgr, wq, bqr)


def kernel(x, edge_index, W1, b1, W2, b2, Wg, bg, Wq, bq):
    ei = edge_index.astype(jnp.int32)
    src, dst = ei[0], ei[1]
    e = src.shape[0]
    chunk = NW * K * CW
    epad = -(-e // chunk) * chunk
    nwpw = epad // (NW * K)
    npad_e = epad - e
    if npad_e:
        pidx = jnp.arange(npad_e, dtype=jnp.int32)
        src = jnp.concatenate([src, (pidx * 97) % N])
        dst = jnp.concatenate([dst, N + pidx % (NPAD - N)])
    src2 = src.reshape(epad // K, K)
    dst2 = dst.reshape(epad // K, K)

    xp = jnp.pad(x, ((0, NPAD - N), (0, 0)))

    degparts = _deg_call(dst2, nwpw)
    p1 = _mm_call(xp, W1)
    d0 = degparts[0][:, None]
    d1 = degparts[1][:, None]
    dinv, ps1 = _scale_call(d0, d1, p1)
    agg1 = _agg_call(src2, dst2, ps1, nwpw)
    ps2 = _layer2_call(agg1, ps1, dinv, b1.reshape(1, D), W2)
    agg2 = _agg_call(src2, dst2, ps2, nwpw)
    gsum = _reduce_call(agg2, ps2, dinv, b2.reshape(1, D))
    q = _head_call(gsum, Wg, bg.reshape(1, D), Wq, bq.reshape(1, NACT))
    return q[None]


# speedup vs baseline: 29.5012x; 1.0204x over previous
import dataclasses
import functools

import jax
import jax.numpy as jnp
from jax import lax
from jax.experimental import pallas as pl
from jax.experimental.pallas import tpu as pltpu
from jax.experimental.pallas import tpu_sc as plsc

N = 10000
NPAD = 10240
D = 128
NACT = 10000
NC = 2
NS = 16
NW = NC * NS
K = 128
NBUF = 2
CW = 16
RPT = NPAD // NS
BM = 512

_MESH = plsc.VectorSubcoreMesh(core_axis_name="c", subcore_axis_name="s")

_CP = pltpu.CompilerParams()
if "needs_layout_passes" in pltpu.CompilerParams.__dataclass_fields__:
    _CP = dataclasses.replace(_CP, needs_layout_passes=False)


def _deg_call(dst2, nwpw):
    @functools.partial(
        pl.kernel,
        out_type=jax.ShapeDtypeStruct((NC, NPAD), jnp.float32),
        mesh=_MESH,
        scratch_types=[
            pltpu.VMEM_SHARED((NS, NPAD), jnp.float32),
            pltpu.VMEM((NPAD,), jnp.float32),
            pltpu.VMEM((nwpw, K), jnp.int32),
            pltpu.VMEM((NS, RPT), jnp.float32),
            pltpu.VMEM((RPT,), jnp.float32),
        ],
        compiler_params=_CP,
    )
    def deg_kernel(dst_hbm, out_hbm, acc_sh, degl, idxd_v, stage_v, res_v):
        c = lax.axis_index("c")
        s = lax.axis_index("s")
        wid = c * NS + s

        @pl.loop(0, NPAD // 16)
        def _(i):
            degl[pl.ds(i * 16, 16)] = jnp.zeros((16,), jnp.float32)

        wbase = pl.multiple_of(wid * nwpw, 8)
        pltpu.sync_copy(dst_hbm.at[pl.ds(wbase, nwpw)], idxd_v)
        ones16 = jnp.ones((16,), jnp.float32)

        @pl.loop(0, nwpw)
        def _(w):
            for j in range(0, K, 16):
                idx16 = idxd_v[w, pl.ds(j, 16)]
                plsc.addupdate_scatter(degl, [idx16], ones16)

        pltpu.sync_copy(degl, acc_sh.at[s])
        plsc.subcore_barrier()
        pltpu.sync_copy(acc_sh.at[:, pl.ds(s * RPT, RPT)], stage_v)

        @pl.loop(0, RPT // 16)
        def _(i):
            acc = jnp.zeros((16,), jnp.float32)
            for t in range(NS):
                acc = acc + stage_v[t, pl.ds(i * 16, 16)]
            res_v[pl.ds(i * 16, 16)] = acc

        pltpu.sync_copy(res_v, out_hbm.at[c, pl.ds(s * RPT, RPT)])

    return deg_kernel(dst2)


def _agg_call(src2, dst2, ps, nwpw):
    @functools.partial(
        pl.kernel,
        out_type=jax.ShapeDtypeStruct((NC, NPAD, D), jnp.float32),
        mesh=_MESH,
        scratch_types=[
            pltpu.VMEM_SHARED((NPAD, D), jnp.float32),
            pltpu.VMEM((CW, K), jnp.int32),
            pltpu.VMEM((CW, K), jnp.int32),
            pltpu.VMEM((NBUF, K, D), jnp.float32),
            pltpu.SemaphoreType.DMA,
            pltpu.SemaphoreType.DMA,
        ],
    )
    def agg_kernel(src_hbm, dst_hbm, ps_hbm, out_hbm,
                   acc_sh, idxs_v, idxd_v, bufs, s0, s1):
        sems = [s0, s1]
        c = lax.axis_index("c")
        s = lax.axis_index("s")
        wid = c * NS + s
        wbase = pl.multiple_of(wid * nwpw, 8)

        @pl.loop(0, K)
        def _(i):
            for j in range(0, D, 16):
                bufs[0, i, pl.ds(j, 16)] = jnp.zeros((16,), jnp.float32)

        for r in range(RPT // K):
            pltpu.sync_copy(bufs.at[0], acc_sh.at[pl.ds(s * RPT + r * K, K)])
        plsc.subcore_barrier()

        @pl.loop(0, nwpw // CW)
        def _(kc):
            cbase = pl.multiple_of(wbase + kc * CW, 8)
            pltpu.sync_copy(src_hbm.at[pl.ds(cbase, CW)], idxs_v)
            pltpu.sync_copy(dst_hbm.at[pl.ds(cbase, CW)], idxd_v)
            for b in range(NBUF):
                pltpu.async_copy(ps_hbm.at[idxs_v.at[b]], bufs.at[b], sems[b])

            @pl.loop(0, CW // NBUF)
            def _(k):
                for b in range(NBUF):
                    w = k * NBUF + b
                    pltpu.make_async_copy(
                        ps_hbm.at[pl.ds(0, K)], bufs.at[b], sems[b]).wait()
                    pltpu.sync_copy(bufs.at[b], acc_sh.at[idxd_v.at[w]],
                                    add=True)
                    wn = w + NBUF

                    @pl.when(wn < CW)
                    def _():
                        pltpu.async_copy(ps_hbm.at[idxs_v.at[wn]], bufs.at[b],
                                         sems[b])

        plsc.subcore_barrier()
        pltpu.sync_copy(acc_sh.at[pl.ds(s * RPT, RPT)],
                        out_hbm.at[c, pl.ds(s * RPT, RPT)])

    return agg_kernel(src2, dst2, ps)


def _mm_scale_body(x_ref, w_ref, d0_ref, d1_ref, dinv_ref, ps_ref):
    p = lax.dot_general(
        x_ref[...], w_ref[...], (((1,), (1,)), ((), ())),
        preferred_element_type=jnp.float32)
    deg = d0_ref[...] + d1_ref[...] + 1.0
    dinv = lax.rsqrt(deg)
    dinv_ref[...] = dinv
    ps_ref[...] = p * dinv


def _mm_scale_call(xp, w, d0, d1):
    return pl.pallas_call(
        _mm_scale_body,
        grid=(NPAD // BM,),
        in_specs=[
            pl.BlockSpec((BM, D), lambda i: (i, 0)),
            pl.BlockSpec((D, D), lambda i: (0, 0)),
            pl.BlockSpec((BM, 1), lambda i: (i, 0)),
            pl.BlockSpec((BM, 1), lambda i: (i, 0)),
        ],
        out_specs=[
            pl.BlockSpec((BM, 1), lambda i: (i, 0)),
            pl.BlockSpec((BM, D), lambda i: (i, 0)),
        ],
        out_shape=[
            jax.ShapeDtypeStruct((NPAD, 1), jnp.float32),
            jax.ShapeDtypeStruct((NPAD, D), jnp.float32),
        ],
    )(xp, w, d0, d1)


def _layer2_body(a0_ref, a1_ref, ps1_ref, dinv_ref, b1_ref, w2_ref, ps2_ref):
    h = dinv_ref[...] * (a0_ref[0] + a1_ref[0] + ps1_ref[...]) + b1_ref[...]
    h = jnp.maximum(h, 0.0)
    p2 = lax.dot_general(h, w2_ref[...], (((1,), (1,)), ((), ())),
                         preferred_element_type=jnp.float32)
    ps2_ref[...] = dinv_ref[...] * p2


def _layer2_call(agg1, ps1, dinv, b1r, w2):
    return pl.pallas_call(
        _layer2_body,
        grid=(NPAD // BM,),
        in_specs=[
            pl.BlockSpec((1, BM, D), lambda i: (0, i, 0)),
            pl.BlockSpec((1, BM, D), lambda i: (1, i, 0)),
            pl.BlockSpec((BM, D), lambda i: (i, 0)),
            pl.BlockSpec((BM, 1), lambda i: (i, 0)),
            pl.BlockSpec((1, D), lambda i: (0, 0)),
            pl.BlockSpec((D, D), lambda i: (0, 0)),
        ],
        out_specs=pl.BlockSpec((BM, D), lambda i: (i, 0)),
        out_shape=jax.ShapeDtypeStruct((NPAD, D), jnp.float32),
    )(agg1, agg1, ps1, dinv, b1r, w2)


def _tail_body(a0_ref, a1_ref, ps2_ref, dinv_ref, b2_ref,
               wg_ref, bg_ref, wq_ref, bq_ref, q_ref, gsum_ref):
    i = pl.program_id(0)
    h = dinv_ref[...] * (a0_ref[0] + a1_ref[0] + ps2_ref[...]) + b2_ref[...]
    h = jnp.maximum(h, 0.0)
    rows = lax.broadcasted_iota(jnp.int32, (BM, 1), 0) + i * BM
    h = jnp.where(rows < N, h, 0.0)

    @pl.when(i == 0)
    def _():
        gsum_ref[...] = jnp.zeros_like(gsum_ref)

    gsum_ref[...] += jnp.sum(h, axis=0, keepdims=True)

    @pl.when(i == NPAD // BM - 1)
    def _():
        g = gsum_ref[...] * (1.0 / N)
        gg = lax.dot_general(g, wg_ref[...], (((1,), (1,)), ((), ())),
                             preferred_element_type=jnp.float32) + bg_ref[...]
        gg = jnp.maximum(gg, 0.0)
        q_ref[...] = lax.dot_general(
            gg, wq_ref[...], (((1,), (1,)), ((), ())),
            preferred_element_type=jnp.float32) + bq_ref[...]


def _tail_call(agg2, ps2, dinv, b2r, wg, bgr, wq, bqr):
    return pl.pallas_call(
        _tail_body,
        grid=(NPAD // BM,),
        in_specs=[
            pl.BlockSpec((1, BM, D), lambda i: (0, i, 0)),
            pl.BlockSpec((1, BM, D), lambda i: (1, i, 0)),
            pl.BlockSpec((BM, D), lambda i: (i, 0)),
            pl.BlockSpec((BM, 1), lambda i: (i, 0)),
            pl.BlockSpec((1, D), lambda i: (0, 0)),
            pl.BlockSpec((D, D), lambda i: (0, 0)),
            pl.BlockSpec((1, D), lambda i: (0, 0)),
            pl.BlockSpec((NACT, D), lambda i: (0, 0)),
            pl.BlockSpec((1, NACT), lambda i: (0, 0)),
        ],
        out_specs=pl.BlockSpec((1, NACT), lambda i: (0, 0)),
        out_shape=jax.ShapeDtypeStruct((1, NACT), jnp.float32),
        scratch_shapes=[pltpu.VMEM((1, D), jnp.float32)],
    )(agg2, agg2, ps2, dinv, b2r, wg, bgr, wq, bqr)


def kernel(x, edge_index, W1, b1, W2, b2, Wg, bg, Wq, bq):
    ei = edge_index.astype(jnp.int32)
    src, dst = ei[0], ei[1]
    e = src.shape[0]
    chunk = NW * K * CW
    epad = -(-e // chunk) * chunk
    nwpw = epad // (NW * K)
    npad_e = epad - e
    if npad_e:
        pidx = jnp.arange(npad_e, dtype=jnp.int32)
        src = jnp.concatenate([src, (pidx * 97) % N])
        dst = jnp.concatenate([dst, N + pidx % (NPAD - N)])
    src2 = src.reshape(epad // K, K)
    dst2 = dst.reshape(epad // K, K)

    xp = jnp.pad(x, ((0, NPAD - N), (0, 0)))

    degparts = _deg_call(dst2, nwpw)
    d0 = degparts[0][:, None]
    d1 = degparts[1][:, None]
    dinv, ps1 = _mm_scale_call(xp, W1, d0, d1)
    agg1 = _agg_call(src2, dst2, ps1, nwpw)
    ps2 = _layer2_call(agg1, ps1, dinv, b1.reshape(1, D), W2)
    agg2 = _agg_call(src2, dst2, ps2, nwpw)
    q = _tail_call(agg2, ps2, dinv, b2.reshape(1, D),
                   Wg, bg.reshape(1, D), Wq, bq.reshape(1, NACT))
    return q[None]


# speedup vs baseline: 29.8237x; 1.0109x over previous
import dataclasses
import functools

import jax
import jax.numpy as jnp
from jax import lax
from jax.experimental import pallas as pl
from jax.experimental.pallas import tpu as pltpu
from jax.experimental.pallas import tpu_sc as plsc

N = 10000
NPAD = 10240
D = 128
NACT = 10000
NC = 2
NS = 16
NW = NC * NS
K = 112
NBUF = 3
CW = 16
RPT = NPAD // NS
BM = 512

_MESH = plsc.VectorSubcoreMesh(core_axis_name="c", subcore_axis_name="s")

_CP = pltpu.CompilerParams()
if "needs_layout_passes" in pltpu.CompilerParams.__dataclass_fields__:
    _CP = dataclasses.replace(_CP, needs_layout_passes=False)


def _deg_call(dst1, eww):
    @functools.partial(
        pl.kernel,
        out_type=jax.ShapeDtypeStruct((NC, NPAD), jnp.float32),
        mesh=_MESH,
        scratch_types=[
            pltpu.VMEM_SHARED((NS, NPAD), jnp.float32),
            pltpu.VMEM((NPAD,), jnp.float32),
            pltpu.VMEM((eww,), jnp.int32),
            pltpu.VMEM((NS, RPT), jnp.float32),
            pltpu.VMEM((RPT,), jnp.float32),
        ],
        compiler_params=_CP,
    )
    def deg_kernel(dst_hbm, out_hbm, acc_sh, degl, idxd_v, stage_v, res_v):
        c = lax.axis_index("c")
        s = lax.axis_index("s")
        wid = c * NS + s

        @pl.loop(0, NPAD // 16)
        def _(i):
            degl[pl.ds(i * 16, 16)] = jnp.zeros((16,), jnp.float32)

        wbase = pl.multiple_of(wid * eww, 8)
        pltpu.sync_copy(dst_hbm.at[pl.ds(wbase, eww)], idxd_v)
        ones16 = jnp.ones((16,), jnp.float32)

        @pl.loop(0, eww // 16)
        def _(i):
            idx16 = idxd_v[pl.ds(i * 16, 16)]
            plsc.addupdate_scatter(degl, [idx16], ones16)

        pltpu.sync_copy(degl, acc_sh.at[s])
        plsc.subcore_barrier()
        pltpu.sync_copy(acc_sh.at[:, pl.ds(s * RPT, RPT)], stage_v)

        @pl.loop(0, RPT // 16)
        def _(i):
            acc = jnp.zeros((16,), jnp.float32)
            for t in range(NS):
                acc = acc + stage_v[t, pl.ds(i * 16, 16)]
            res_v[pl.ds(i * 16, 16)] = acc

        pltpu.sync_copy(res_v, out_hbm.at[c, pl.ds(s * RPT, RPT)])

    return deg_kernel(dst1)


def _agg_call(src2, dst2, ps, nwpw):
    @functools.partial(
        pl.kernel,
        out_type=jax.ShapeDtypeStruct((NC, NPAD, D), jnp.float32),
        mesh=_MESH,
        scratch_types=[
            pltpu.VMEM_SHARED((NPAD, D), jnp.float32),
            pltpu.VMEM((CW, K), jnp.int32),
            pltpu.VMEM((CW, K), jnp.int32),
            pltpu.VMEM((NBUF, K, D), jnp.float32),
            pltpu.SemaphoreType.DMA,
            pltpu.SemaphoreType.DMA,
            pltpu.SemaphoreType.DMA,
        ],
    )
    def agg_kernel(src_hbm, dst_hbm, ps_hbm, out_hbm,
                   acc_sh, idxs_v, idxd_v, bufs, s0, s1, s2):
        sems = [s0, s1, s2]
        c = lax.axis_index("c")
        s = lax.axis_index("s")
        wid = c * NS + s
        wbase = pl.multiple_of(wid * nwpw, 8)

        @pl.loop(0, K)
        def _(i):
            for j in range(0, D, 16):
                bufs[0, i, pl.ds(j, 16)] = jnp.zeros((16,), jnp.float32)

        full, rem = RPT // K, RPT % K
        for r in range(full):
            pltpu.sync_copy(bufs.at[0], acc_sh.at[pl.ds(s * RPT + r * K, K)])
        if rem:
            pltpu.sync_copy(bufs.at[0, pl.ds(0, rem)],
                            acc_sh.at[pl.ds(s * RPT + full * K, rem)])
        plsc.subcore_barrier()

        @pl.loop(0, nwpw // CW)
        def _(kc):
            cbase = pl.multiple_of(wbase + kc * CW, 8)
            pltpu.sync_copy(src_hbm.at[pl.ds(cbase, CW)], idxs_v)
            pltpu.sync_copy(dst_hbm.at[pl.ds(cbase, CW)], idxd_v)
            for b in range(NBUF):
                pltpu.async_copy(ps_hbm.at[idxs_v.at[b]], bufs.at[b], sems[b])

            for w in range(CW):
                b = w % NBUF
                pltpu.make_async_copy(
                    ps_hbm.at[pl.ds(0, K)], bufs.at[b], sems[b]).wait()
                pltpu.sync_copy(bufs.at[b], acc_sh.at[idxd_v.at[w]],
                                add=True)
                wn = w + NBUF
                if wn < CW:
                    pltpu.async_copy(ps_hbm.at[idxs_v.at[wn]], bufs.at[b],
                                     sems[b])

        plsc.subcore_barrier()
        pltpu.sync_copy(acc_sh.at[pl.ds(s * RPT, RPT)],
                        out_hbm.at[c, pl.ds(s * RPT, RPT)])

    return agg_kernel(src2, dst2, ps)


def _mm_scale_body(x_ref, w_ref, d0_ref, d1_ref, dinv_ref, ps_ref):
    p = lax.dot_general(
        x_ref[...], w_ref[...], (((1,), (1,)), ((), ())),
        preferred_element_type=jnp.float32)
    deg = d0_ref[...] + d1_ref[...] + 1.0
    dinv = lax.rsqrt(deg)
    dinv_ref[...] = dinv
    ps_ref[...] = p * dinv


def _mm_scale_call(xp, w, d0, d1):
    return pl.pallas_call(
        _mm_scale_body,
        grid=(NPAD // BM,),
        in_specs=[
            pl.BlockSpec((BM, D), lambda i: (i, 0)),
            pl.BlockSpec((D, D), lambda i: (0, 0)),
            pl.BlockSpec((BM, 1), lambda i: (i, 0)),
            pl.BlockSpec((BM, 1), lambda i: (i, 0)),
        ],
        out_specs=[
            pl.BlockSpec((BM, 1), lambda i: (i, 0)),
            pl.BlockSpec((BM, D), lambda i: (i, 0)),
        ],
        out_shape=[
            jax.ShapeDtypeStruct((NPAD, 1), jnp.float32),
            jax.ShapeDtypeStruct((NPAD, D), jnp.float32),
        ],
    )(xp, w, d0, d1)


def _layer2_body(a0_ref, a1_ref, ps1_ref, dinv_ref, b1_ref, w2_ref, ps2_ref):
    h = dinv_ref[...] * (a0_ref[0] + a1_ref[0] + ps1_ref[...]) + b1_ref[...]
    h = jnp.maximum(h, 0.0)
    p2 = lax.dot_general(h, w2_ref[...], (((1,), (1,)), ((), ())),
                         preferred_element_type=jnp.float32)
    ps2_ref[...] = dinv_ref[...] * p2


def _layer2_call(agg1, ps1, dinv, b1r, w2):
    return pl.pallas_call(
        _layer2_body,
        grid=(NPAD // BM,),
        in_specs=[
            pl.BlockSpec((1, BM, D), lambda i: (0, i, 0)),
            pl.BlockSpec((1, BM, D), lambda i: (1, i, 0)),
            pl.BlockSpec((BM, D), lambda i: (i, 0)),
            pl.BlockSpec((BM, 1), lambda i: (i, 0)),
            pl.BlockSpec((1, D), lambda i: (0, 0)),
            pl.BlockSpec((D, D), lambda i: (0, 0)),
        ],
        out_specs=pl.BlockSpec((BM, D), lambda i: (i, 0)),
        out_shape=jax.ShapeDtypeStruct((NPAD, D), jnp.float32),
    )(agg1, agg1, ps1, dinv, b1r, w2)


def _tail_body(a0_ref, a1_ref, ps2_ref, dinv_ref, b2_ref,
               wg_ref, bg_ref, wq_ref, bq_ref, q_ref, gsum_ref):
    i = pl.program_id(0)
    h = dinv_ref[...] * (a0_ref[0] + a1_ref[0] + ps2_ref[...]) + b2_ref[...]
    h = jnp.maximum(h, 0.0)
    rows = lax.broadcasted_iota(jnp.int32, (BM, 1), 0) + i * BM
    h = jnp.where(rows < N, h, 0.0)

    @pl.when(i == 0)
    def _():
        gsum_ref[...] = jnp.zeros_like(gsum_ref)

    gsum_ref[...] += jnp.sum(h, axis=0, keepdims=True)

    @pl.when(i == NPAD // BM - 1)
    def _():
        g = gsum_ref[...] * (1.0 / N)
        gg = lax.dot_general(g, wg_ref[...], (((1,), (1,)), ((), ())),
                             preferred_element_type=jnp.float32) + bg_ref[...]
        gg = jnp.maximum(gg, 0.0)
        q_ref[...] = lax.dot_general(
            gg, wq_ref[...], (((1,), (1,)), ((), ())),
            preferred_element_type=jnp.float32) + bq_ref[...]


def _tail_call(agg2, ps2, dinv, b2r, wg, bgr, wq, bqr):
    return pl.pallas_call(
        _tail_body,
        grid=(NPAD // BM,),
        in_specs=[
            pl.BlockSpec((1, BM, D), lambda i: (0, i, 0)),
            pl.BlockSpec((1, BM, D), lambda i: (1, i, 0)),
            pl.BlockSpec((BM, D), lambda i: (i, 0)),
            pl.BlockSpec((BM, 1), lambda i: (i, 0)),
            pl.BlockSpec((1, D), lambda i: (0, 0)),
            pl.BlockSpec((D, D), lambda i: (0, 0)),
            pl.BlockSpec((1, D), lambda i: (0, 0)),
            pl.BlockSpec((NACT, D), lambda i: (0, 0)),
            pl.BlockSpec((1, NACT), lambda i: (0, 0)),
        ],
        out_specs=pl.BlockSpec((1, NACT), lambda i: (0, 0)),
        out_shape=jax.ShapeDtypeStruct((1, NACT), jnp.float32),
        scratch_shapes=[pltpu.VMEM((1, D), jnp.float32)],
    )(agg2, agg2, ps2, dinv, b2r, wg, bgr, wq, bqr)


def kernel(x, edge_index, W1, b1, W2, b2, Wg, bg, Wq, bq):
    ei = edge_index.astype(jnp.int32)
    src, dst = ei[0], ei[1]
    e = src.shape[0]
    chunk = NW * K * CW
    epad = -(-e // chunk) * chunk
    nwpw = epad // (NW * K)
    npad_e = epad - e
    if npad_e:
        pidx = jnp.arange(npad_e, dtype=jnp.int32)
        src = jnp.concatenate([src, (pidx * 97) % N])
        dst = jnp.concatenate([dst, N + pidx % (NPAD - N)])
    src2 = src.reshape(epad // K, K)
    dst2 = dst.reshape(epad // K, K)

    xp = jnp.pad(x, ((0, NPAD - N), (0, 0)))

    degparts = _deg_call(dst, epad // NW)
    d0 = degparts[0][:, None]
    d1 = degparts[1][:, None]
    dinv, ps1 = _mm_scale_call(xp, W1, d0, d1)
    agg1 = _agg_call(src2, dst2, ps1, nwpw)
    ps2 = _layer2_call(agg1, ps1, dinv, b1.reshape(1, D), W2)
    agg2 = _agg_call(src2, dst2, ps2, nwpw)
    q = _tail_call(agg2, ps2, dinv, b2.reshape(1, D),
                   Wg, bg.reshape(1, D), Wq, bq.reshape(1, NACT))
    return q[None]
